# baseline (speedup 1.0000x reference)
import functools

import jax
import jax.numpy as jnp
from jax import lax
from jax.experimental import pallas as pl
from jax.experimental.pallas import tpu as pltpu, tpu_sc as plsc

_NC = 2
_NS = 16
_L = 16
_NW = _NC * _NS


def _make_kernel(B, V, D):
    assert B % (8 * _NW) == 0
    b_per_w = B // _NW
    mesh = plsc.VectorSubcoreMesh(core_axis_name="c", subcore_axis_name="s")

    @functools.partial(
        pl.kernel,
        out_type=jax.ShapeDtypeStruct((D, B), jnp.float32),
        mesh=mesh,
        scratch_types=[
            pltpu.VMEM((b_per_w,), jnp.int32),
            pltpu.VMEM((D * b_per_w,), jnp.int32),
            pltpu.VMEM((D * b_per_w,), jnp.float32),
            pltpu.SemaphoreType.DMA,
        ],
        compiler_params=pltpu.CompilerParams(use_tc_tiling_on_sc=False),
    )
    def k(idx_hbm, tab_hbm, out_hbm, idx_v, eidx_v, dst_v, sem):
        wid = lax.axis_index("s") * _NC + lax.axis_index("c")
        base = wid * b_per_w
        pltpu.sync_copy(idx_hbm.at[pl.ds(base, b_per_w)], idx_v)

        def body(j, _):
            start = pl.multiple_of(j * _L, _L)
            v = idx_v[pl.ds(start, _L)]
            for c in range(D):
                eidx_v[pl.ds(c * b_per_w + start, _L)] = v + c * V
            return 0

        lax.fori_loop(0, b_per_w // _L, body, 0)
        pltpu.async_copy(tab_hbm.at[eidx_v], dst_v, sem).wait()
        handles = [
            pltpu.async_copy(
                dst_v.at[pl.ds(c * b_per_w, b_per_w)],
                out_hbm.at[c, pl.ds(base, b_per_w)],
                sem,
            )
            for c in range(D)
        ]
        for h in handles:
            h.wait()

    return k


@jax.jit
def kernel(user_fea, embedding_location):
    B, _ = user_fea.shape
    V, D = embedding_location.shape
    idx = user_fea[:, 0].astype(jnp.int32)
    tab_flat = embedding_location.T.reshape(-1)
    k = _make_kernel(B, V, D)
    out_t = k(idx, tab_flat)
    return out_t.T


# speedup vs baseline: 4.9219x; 4.9219x over previous
import functools

import jax
import jax.numpy as jnp
from jax import lax
from jax.experimental import pallas as pl
from jax.experimental.pallas import tpu as pltpu, tpu_sc as plsc

_NC = 2
_NS = 16
_L = 16
_NW = _NC * _NS


def _make_kernel(B, V, D):
    assert B % (8 * _NW) == 0 and D % _L == 0
    b_per_w = B // _NW
    mesh = plsc.VectorSubcoreMesh(core_axis_name="c", subcore_axis_name="s")

    @functools.partial(
        pl.kernel,
        out_type=jax.ShapeDtypeStruct((B, D), jnp.float32),
        mesh=mesh,
        scratch_types=[
            pltpu.VMEM((b_per_w,), jnp.int32),
            pltpu.VMEM((b_per_w, D), jnp.float32),
            pltpu.SemaphoreType.DMA,
        ],
        compiler_params=pltpu.CompilerParams(use_tc_tiling_on_sc=False),
    )
    def k(idx_hbm, table_hbm, out_hbm, idx_v, rows_v, sem):
        wid = lax.axis_index("s") * _NC + lax.axis_index("c")
        base = wid * b_per_w
        pltpu.sync_copy(idx_hbm.at[pl.ds(base, b_per_w)], idx_v)
        pltpu.async_copy(table_hbm.at[idx_v], rows_v, sem).wait()
        pltpu.sync_copy(rows_v, out_hbm.at[pl.ds(base, b_per_w)])

    return k


@jax.jit
def kernel(user_fea, embedding_location):
    B, _ = user_fea.shape
    V, D = embedding_location.shape
    idx = user_fea[:, 0].astype(jnp.int32)
    k = _make_kernel(B, V, D)
    return k(idx, embedding_location)


# speedup vs baseline: 16.9237x; 3.4384x over previous
import functools

import jax
import jax.numpy as jnp
from jax import lax
from jax.experimental import pallas as pl
from jax.experimental.pallas import tpu as pltpu, tpu_sc as plsc

_NC = 2
_NS = 16
_L = 16
_NW = _NC * _NS
_G = 16


def _make_kernel(B, V, D):
    assert B % (8 * _NW) == 0 and D % _L == 0
    b_per_w = B // _NW
    mesh = plsc.VectorSubcoreMesh(core_axis_name="c", subcore_axis_name="s")

    @functools.partial(
        pl.kernel,
        out_type=jax.ShapeDtypeStruct((B, D), jnp.float32),
        mesh=mesh,
        scratch_types=[
            pltpu.VMEM((b_per_w,), jnp.int32),
            pltpu.VMEM((_G, D, 128), jnp.float32),
            pltpu.VMEM((_G, D), jnp.float32),
            pltpu.SemaphoreType.DMA,
        ],
        compiler_params=pltpu.CompilerParams(
            use_tc_tiling_on_sc=True,
            needs_layout_passes=False,
        ),
    )
    def k(idx_hbm, tab_hbm, out_hbm, idx_v, tbufs, obuf, sem):
        wid = lax.axis_index("s") * _NC + lax.axis_index("c")
        base = wid * b_per_w
        pltpu.sync_copy(idx_hbm.at[pl.ds(base, b_per_w)], idx_v)
        lane = lax.iota(jnp.int32, _L)

        def group(g, _):
            i0 = g * _G
            start = pl.multiple_of(i0, _L)
            chunk = idx_v[pl.ds(start, _L)]
            handles = []
            for gi in range(_G):
                t = chunk[gi] >> 7
                off = pl.multiple_of(t * 128, 128)
                for a in range(D // 8):
                    handles.append(
                        pltpu.async_copy(
                            tab_hbm.at[pl.ds(8 * a, 8), pl.ds(off, 128)],
                            tbufs.at[gi, pl.ds(8 * a, 8), :],
                            sem,
                        )
                    )
            for h in handles:
                h.wait()
            for gi in range(_G):
                lv = jnp.zeros((_L,), jnp.int32) + (chunk[gi] & 127)
                for h2 in range(D // _L):
                    rows = lane + h2 * _L
                    vals = plsc.load_gather(tbufs.at[gi], [rows, lv])
                    obuf[gi, pl.ds(h2 * _L, _L)] = vals
            pltpu.sync_copy(obuf, out_hbm.at[pl.ds(base + i0, _G), :])
            return 0

        lax.fori_loop(0, b_per_w // _G, group, 0)

    return k


@jax.jit
def kernel(user_fea, embedding_location):
    B, _ = user_fea.shape
    V, D = embedding_location.shape
    idx = user_fea[:, 0].astype(jnp.int32)
    k = _make_kernel(B, V, D)
    return k(idx, embedding_location.T)


# speedup vs baseline: 18.0012x; 1.0637x over previous
import functools

import jax
import jax.numpy as jnp
from jax import lax
from jax.experimental import pallas as pl
from jax.experimental.pallas import tpu as pltpu, tpu_sc as plsc

_NC = 2
_NS = 16
_L = 16
_NW = _NC * _NS
_G = 8


def _make_kernel(B, V, D):
    assert B % (8 * _NW) == 0 and D % _L == 0
    b_per_w = B // _NW
    n_pairs = b_per_w // (2 * _G)
    mesh = plsc.VectorSubcoreMesh(core_axis_name="c", subcore_axis_name="s")

    @functools.partial(
        pl.kernel,
        out_type=jax.ShapeDtypeStruct((B, D), jnp.float32),
        mesh=mesh,
        scratch_types=[
            pltpu.VMEM((b_per_w + _L,), jnp.int32),
            pltpu.VMEM((2, _G, D, 128), jnp.float32),
            pltpu.VMEM((_G, D), jnp.float32),
            pltpu.SemaphoreType.DMA,
            pltpu.SemaphoreType.DMA,
        ],
        compiler_params=pltpu.CompilerParams(
            use_tc_tiling_on_sc=True,
            needs_layout_passes=False,
        ),
    )
    def k(idx_hbm, tab_hbm, out_hbm, idx_v, tb, ob, sem0, sem1):
        wid = lax.axis_index("s") * _NC + lax.axis_index("c")
        base = wid * b_per_w
        pltpu.sync_copy(idx_hbm.at[pl.ds(base, b_per_w)], idx_v.at[pl.ds(0, b_per_w)])
        lane = lax.iota(jnp.int32, _L)

        def chunk_of(h):
            start = pl.multiple_of(h * _G, _G)
            return idx_v[pl.ds(start, _L)]

        def fire(h, bi, sem):
            ch = chunk_of(h)
            for gi in range(_G):
                off = pl.multiple_of((ch[gi] >> 7) * 128, 128)
                pltpu.async_copy(
                    tab_hbm.at[:, pl.ds(off, 128)], tb.at[bi, gi], sem
                )

        def drain(bi, sem):
            for gi in range(_G):
                pltpu.make_async_copy(
                    tab_hbm.at[:, pl.ds(0, 128)], tb.at[bi, gi], sem
                ).wait()

        def extract_store(h, bi):
            ch = chunk_of(h)
            for gi in range(_G):
                lv = jnp.zeros((_L,), jnp.int32) + (ch[gi] & 127)
                for h2 in range(D // _L):
                    vals = plsc.load_gather(tb.at[bi, gi], [lane + h2 * _L, lv])
                    ob[gi, pl.ds(h2 * _L, _L)] = vals
            pltpu.sync_copy(ob, out_hbm.at[pl.ds(base + h * _G, _G), :])

        fire(0, 0, sem0)

        def pair(p, _):
            fire(2 * p + 1, 1, sem1)
            drain(0, sem0)
            extract_store(2 * p, 0)

            @pl.when(p < n_pairs - 1)
            def _():
                fire(2 * p + 2, 0, sem0)

            drain(1, sem1)
            extract_store(2 * p + 1, 1)
            return 0

        lax.fori_loop(0, n_pairs, pair, 0)

    return k


@jax.jit
def kernel(user_fea, embedding_location):
    B, _ = user_fea.shape
    V, D = embedding_location.shape
    idx = user_fea[:, 0].astype(jnp.int32)
    k = _make_kernel(B, V, D)
    return k(idx, embedding_location.T)


# speedup vs baseline: 18.0797x; 1.0044x over previous
import functools

import jax
import jax.numpy as jnp
from jax import lax
from jax.experimental import pallas as pl
from jax.experimental.pallas import tpu as pltpu, tpu_sc as plsc

_NC = 2
_NS = 16
_L = 16
_NW = _NC * _NS
_G = 8


def _make_kernel(B, V, D):
    assert B % (8 * _NW) == 0 and D % _L == 0
    b_per_w = B // _NW
    n_pairs = b_per_w // (2 * _G)
    mesh = plsc.VectorSubcoreMesh(core_axis_name="c", subcore_axis_name="s")

    @functools.partial(
        pl.kernel,
        out_type=jax.ShapeDtypeStruct((B, D), jnp.float32),
        mesh=mesh,
        scratch_types=[
            pltpu.VMEM((b_per_w + _L,), jnp.int32),
            pltpu.VMEM((2, _G, D, 128), jnp.float32),
            pltpu.VMEM((2, _G, D), jnp.float32),
            pltpu.SemaphoreType.DMA,
            pltpu.SemaphoreType.DMA,
            pltpu.SemaphoreType.DMA,
        ],
        compiler_params=pltpu.CompilerParams(
            use_tc_tiling_on_sc=True,
            needs_layout_passes=False,
        ),
    )
    def k(idx_hbm, tab_hbm, out_hbm, idx_v, tb, ob, sem0, sem1, osem):
        wid = lax.axis_index("s") * _NC + lax.axis_index("c")
        base = wid * b_per_w
        pltpu.sync_copy(idx_hbm.at[pl.ds(base, b_per_w)], idx_v.at[pl.ds(0, b_per_w)])
        lane = lax.iota(jnp.int32, _L)

        def chunk_of(h):
            start = pl.multiple_of(h * _G, _G)
            return idx_v[pl.ds(start, _L)]

        def fire(h, bi, sem):
            ch = chunk_of(h)
            for gi in range(_G):
                off = pl.multiple_of((ch[gi] >> 7) * 128, 128)
                pltpu.async_copy(
                    tab_hbm.at[:, pl.ds(off, 128)], tb.at[bi, gi], sem
                )

        def drain(bi, sem):
            for gi in range(_G):
                pltpu.make_async_copy(
                    tab_hbm.at[:, pl.ds(0, 128)], tb.at[bi, gi], sem
                ).wait()

        def drain_out(bi):
            pltpu.make_async_copy(
                ob.at[bi], out_hbm.at[pl.ds(base, _G), :], osem
            ).wait()

        def extract_store(h, bi, first):
            ch = chunk_of(h)
            if not first:
                drain_out(bi)
            for gi in range(_G):
                lv = jnp.zeros((_L,), jnp.int32) + (ch[gi] & 127)
                for h2 in range(D // _L):
                    vals = plsc.load_gather(tb.at[bi, gi], [lane + h2 * _L, lv])
                    ob[bi, gi, pl.ds(h2 * _L, _L)] = vals
            pltpu.async_copy(ob.at[bi], out_hbm.at[pl.ds(base + h * _G, _G), :], osem)

        fire(0, 0, sem0)
        fire(1, 1, sem1)
        drain(0, sem0)
        extract_store(0, 0, True)
        fire(2, 0, sem0)
        drain(1, sem1)
        extract_store(1, 1, True)

        def pair(p, _):
            fire(2 * p + 3, 1, sem1)
            drain(0, sem0)
            extract_store(2 * p + 2, 0, False)

            @pl.when(p < n_pairs - 2)
            def _():
                fire(2 * p + 4, 0, sem0)

            drain(1, sem1)
            extract_store(2 * p + 3, 1, False)
            return 0

        lax.fori_loop(0, n_pairs - 1, pair, 0)
        drain_out(0)
        drain_out(1)

    return k


@jax.jit
def kernel(user_fea, embedding_location):
    B, _ = user_fea.shape
    V, D = embedding_location.shape
    idx = user_fea[:, 0].astype(jnp.int32)
    k = _make_kernel(B, V, D)
    return k(idx, embedding_location.T)
